# Initial kernel scaffold; baseline (speedup 1.0000x reference)
import jax
import jax.numpy as jnp
from jax.experimental import pallas as pl


def kernel(q_fine, q_coarse, M, cb_fine, cb_coarse):
    raise NotImplementedError("write your pallas kernel here")


# speedup vs baseline: 3.9488x; 3.9488x over previous
import functools

import jax
import jax.numpy as jnp
import numpy as np
from jax import lax
from jax.experimental import pallas as pl
from jax.experimental.pallas import tpu as pltpu
from jax.experimental.pallas import tpu_sc as plsc

NEIGHBOR_PROB = 0.05
STRENGTH = 0.5
K = 1024
D = 256
_SKIP = int(np.ceil(0.05 * K))
POS = int(np.clip(_SKIP + int(np.floor(STRENGTH * (K - _SKIP - 1))), 0, K - 1))
INT_MIN = np.int32(-2147483648)
INT_MAX = np.int32(2147483647)


def _mask_body(m_ref, of_ref, oc_ref):
    m = m_ref[0]
    def pool(fh, fw, oh, ow):
        ih = jax.lax.broadcasted_iota(jnp.int32, (oh, 128), 0)
        rh = jax.lax.broadcasted_iota(jnp.int32, (oh, 128), 1)
        Ph = (rh // fh == ih).astype(jnp.float32)
        rw = jax.lax.broadcasted_iota(jnp.int32, (256, ow), 0)
        iw = jax.lax.broadcasted_iota(jnp.int32, (256, ow), 1)
        Pw = (rw // fw == iw).astype(jnp.float32)
        s1 = jax.lax.dot_general(Ph, m, (((1,), (0,)), ((), ())),
                                 preferred_element_type=jnp.float32)
        s2 = jax.lax.dot_general(s1, Pw, (((1,), (0,)), ((), ())),
                                 preferred_element_type=jnp.float32)
        return (s2 > 0).astype(jnp.float32)
    of_ref[0] = pool(2, 2, 64, 128)
    oc_ref[0] = pool(4, 4, 32, 64)


def _project_masks(M):
    Mf = M.astype(jnp.float32).reshape(4, 128, 256)
    return pl.pallas_call(
        _mask_body,
        grid=(4,),
        in_specs=[pl.BlockSpec((1, 128, 256), lambda i: (i, 0, 0))],
        out_specs=[pl.BlockSpec((1, 64, 128), lambda i: (i, 0, 0)),
                   pl.BlockSpec((1, 32, 64), lambda i: (i, 0, 0))],
        out_shape=[jax.ShapeDtypeStruct((4, 64, 128), jnp.float32),
                   jax.ShapeDtypeStruct((4, 32, 64), jnp.float32)],
    )(Mf)


def _matmul_body(zf_ref, cbt_ref, g_ref):
    g_ref[...] = jax.lax.dot_general(
        zf_ref[...], cbt_ref[...], (((1,), (0,)), ((), ())),
        preferred_element_type=jnp.float32)


def _distances(zf, cb_t, T):
    N = zf.shape[0]
    NB = N // T
    return pl.pallas_call(
        _matmul_body,
        grid=(NB,),
        in_specs=[pl.BlockSpec((T, D), lambda i: (i, 0)),
                  pl.BlockSpec((D, K), lambda i: (0, 0))],
        out_specs=pl.BlockSpec((T, K), lambda i: (i, 0)),
        out_shape=jax.ShapeDtypeStruct((N, K), jnp.float32),
    )(zf, cb_t)


def _select_body(d2_ref, nb_ref, m_ref, out_ref, *, T):
    d2 = d2_ref[...]
    ib = jax.lax.bitcast_convert_type(d2, jnp.int32)
    keys = jnp.where(ib < 0, ib ^ INT_MAX, ib)
    kiota = jax.lax.broadcasted_iota(jnp.int32, (T, K), 1)

    kmin = jnp.min(keys, axis=1, keepdims=True)
    i1 = jnp.min(jnp.where(keys == kmin, kiota, jnp.int32(K)),
                 axis=1, keepdims=True)
    keys2 = jnp.where(kiota == i1, INT_MAX, keys)
    kmin2 = jnp.min(keys2, axis=1, keepdims=True)
    i2 = jnp.min(jnp.where(keys2 == kmin2, kiota, jnp.int32(K)),
                 axis=1, keepdims=True)

    def vstep(i, v):
        t = v | jnp.left_shift(jnp.int32(1), 31 - i)
        s = t ^ INT_MIN
        cnt = jnp.sum((keys < s).astype(jnp.int32), axis=1, keepdims=True)
        return jnp.where(cnt <= POS, t, v)
    v = jax.lax.fori_loop(0, 32, vstep, jnp.zeros((T, 1), jnp.int32))
    vk = v ^ INT_MIN
    cless = jnp.sum((keys < vk).astype(jnp.int32), axis=1, keepdims=True)
    r = POS - cless
    eq = keys == vk

    def istep(i, j):
        t = j + jnp.left_shift(jnp.int32(1), 9 - i)
        cnt = jnp.sum((eq & (kiota < t)).astype(jnp.int32),
                      axis=1, keepdims=True)
        return jnp.where(cnt <= r, t, j)
    chosen = jax.lax.fori_loop(0, 10, istep, jnp.zeros((T, 1), jnp.int32))

    code = jnp.where(nb_ref[0] != 0, i2, chosen)
    tbase = pl.program_id(0) * T + K
    tiota = jax.lax.broadcasted_iota(jnp.int32, (T, 1), 0)
    out_ref[0] = jnp.where(m_ref[0] != 0, code, tbase + tiota)


def _select_codes(d2, nb3, m3, T):
    N = d2.shape[0]
    NB = N // T
    return pl.pallas_call(
        functools.partial(_select_body, T=T),
        grid=(NB,),
        in_specs=[pl.BlockSpec((T, K), lambda i: (i, 0)),
                  pl.BlockSpec((1, T, 1), lambda i: (i, 0, 0)),
                  pl.BlockSpec((1, T, 1), lambda i: (i, 0, 0))],
        out_specs=pl.BlockSpec((1, T, 1), lambda i: (i, 0, 0)),
        out_shape=jax.ShapeDtypeStruct((NB, T, 1), jnp.int32),
    )(d2, nb3, m3)


def _make_sc_gather(B, CHUNK):
    mesh = plsc.VectorSubcoreMesh(core_axis_name="c", subcore_axis_name="s")
    b_per_w = B // 32

    @functools.partial(
        pl.kernel, mesh=mesh,
        out_type=jax.ShapeDtypeStruct((B, D), jnp.float32),
        scratch_types=[pltpu.VMEM((CHUNK,), jnp.int32),
                       pltpu.VMEM((CHUNK, D), jnp.float32),
                       pltpu.SemaphoreType.DMA],
    )
    def k(table_hbm, idx_hbm, out_hbm, idx_v, rows_v, sem):
        wid = lax.axis_index("s") * 2 + lax.axis_index("c")
        base = wid * b_per_w
        for j in range(b_per_w // CHUNK):
            off = base + j * CHUNK
            pltpu.sync_copy(idx_hbm.at[pl.ds(off, CHUNK)], idx_v)
            pltpu.async_copy(table_hbm.at[idx_v], rows_v, sem).wait()
            pltpu.sync_copy(rows_v, out_hbm.at[pl.ds(off, CHUNK)])

    return k


def _one_scale(q, cb, mask, rng_key, T):
    B, Dd, H, W = q.shape
    N = B * H * W
    NB = N // T
    ztr = jnp.transpose(q, (0, 2, 3, 1))
    zf = ztr.reshape(-1, Dd)
    G = _distances(zf, cb.T, T)
    ztr_b, zf_b, cb_b, G_b = jax.lax.optimization_barrier((ztr, zf, cb, G))
    if H * W >= 8192:
        zsq = jax.lax.optimization_barrier(
            jnp.sum(zf_b * zf_b, axis=1, keepdims=True))
    else:
        zsq = jax.lax.optimization_barrier(
            jnp.sum(ztr_b * ztr_b, axis=3)).reshape(-1, 1)
    csq = jax.lax.optimization_barrier(jnp.sum(cb_b * cb_b, axis=1))
    d2 = (zsq - 2.0 * G_b) + csq[None, :]
    d2 = jax.lax.optimization_barrier(d2)
    use_nb = jax.random.uniform(rng_key, (zf.shape[0],)) < NEIGHBOR_PROB

    nb3 = use_nb.astype(jnp.int32).reshape(NB, T, 1)
    m3 = mask.astype(jnp.int32).reshape(NB, T, 1)
    codes = _select_codes(d2, nb3, m3, T).reshape(N)
    table = jnp.concatenate([cb, zf], axis=0)
    out_tok = _make_sc_gather(N, 128)(table, codes)
    return out_tok.reshape(B, H, W, Dd).transpose(0, 3, 1, 2)


def kernel(q_fine, q_coarse, M, cb_fine, cb_coarse):
    mask_f, mask_c = _project_masks(M)
    key = jax.random.key(42)
    kf, kc = jax.random.split(key)
    q_coarse_a = _one_scale(q_coarse, cb_coarse, mask_c, kc, 512)
    q_fine_a = _one_scale(q_fine, cb_fine, mask_f, kf, 512)
    return (q_fine_a, q_coarse_a)


# speedup vs baseline: 4.2568x; 1.0780x over previous
import functools

import jax
import jax.numpy as jnp
import numpy as np
from jax import lax
from jax.experimental import pallas as pl
from jax.experimental.pallas import tpu as pltpu
from jax.experimental.pallas import tpu_sc as plsc

NEIGHBOR_PROB = 0.05
STRENGTH = 0.5
K = 1024
D = 256
_SKIP = int(np.ceil(0.05 * K))
POS = int(np.clip(_SKIP + int(np.floor(STRENGTH * (K - _SKIP - 1))), 0, K - 1))
INT_MIN = np.int32(-2147483648)
INT_MAX = np.int32(2147483647)


def _mask_body(m_ref, of_ref, oc_ref):
    m = m_ref[0]
    def pool(fh, fw, oh, ow):
        ih = jax.lax.broadcasted_iota(jnp.int32, (oh, 128), 0)
        rh = jax.lax.broadcasted_iota(jnp.int32, (oh, 128), 1)
        Ph = (rh // fh == ih).astype(jnp.float32)
        rw = jax.lax.broadcasted_iota(jnp.int32, (256, ow), 0)
        iw = jax.lax.broadcasted_iota(jnp.int32, (256, ow), 1)
        Pw = (rw // fw == iw).astype(jnp.float32)
        s1 = jax.lax.dot_general(Ph, m, (((1,), (0,)), ((), ())),
                                 preferred_element_type=jnp.float32)
        s2 = jax.lax.dot_general(s1, Pw, (((1,), (0,)), ((), ())),
                                 preferred_element_type=jnp.float32)
        return (s2 > 0).astype(jnp.float32)
    of_ref[0] = pool(2, 2, 64, 128)
    oc_ref[0] = pool(4, 4, 32, 64)


def _project_masks(M):
    Mf = M.astype(jnp.float32).reshape(4, 128, 256)
    return pl.pallas_call(
        _mask_body,
        grid=(4,),
        in_specs=[pl.BlockSpec((1, 128, 256), lambda i: (i, 0, 0))],
        out_specs=[pl.BlockSpec((1, 64, 128), lambda i: (i, 0, 0)),
                   pl.BlockSpec((1, 32, 64), lambda i: (i, 0, 0))],
        out_shape=[jax.ShapeDtypeStruct((4, 64, 128), jnp.float32),
                   jax.ShapeDtypeStruct((4, 32, 64), jnp.float32)],
    )(Mf)


def _select_body(zf_ref, cbt_ref, csq_ref, zsq_ref, nb_ref, m_ref, out_ref,
                 *, T):
    G = jax.lax.dot_general(
        zf_ref[...], cbt_ref[...], (((1,), (0,)), ((), ())),
        preferred_element_type=jnp.float32)
    d2 = (zsq_ref[0] - 2.0 * G) + csq_ref[...]
    ib = jax.lax.bitcast_convert_type(d2, jnp.int32)
    keys = jnp.where(ib < 0, ib ^ INT_MAX, ib)
    kiota = jax.lax.broadcasted_iota(jnp.int32, (T, K), 1)

    kmin = jnp.min(keys, axis=1, keepdims=True)
    i1 = jnp.min(jnp.where(keys == kmin, kiota, jnp.int32(K)),
                 axis=1, keepdims=True)
    keys2 = jnp.where(kiota == i1, INT_MAX, keys)
    kmin2 = jnp.min(keys2, axis=1, keepdims=True)
    i2 = jnp.min(jnp.where(keys2 == kmin2, kiota, jnp.int32(K)),
                 axis=1, keepdims=True)

    def vstep(i, v):
        t = v | jnp.left_shift(jnp.int32(1), 31 - i)
        s = t ^ INT_MIN
        cnt = jnp.sum((keys < s).astype(jnp.int32), axis=1, keepdims=True)
        return jnp.where(cnt <= POS, t, v)
    v = jax.lax.fori_loop(0, 32, vstep, jnp.zeros((T, 1), jnp.int32))
    vk = v ^ INT_MIN
    cless = jnp.sum((keys < vk).astype(jnp.int32), axis=1, keepdims=True)
    r = POS - cless
    eq = keys == vk

    def istep(i, j):
        t = j + jnp.left_shift(jnp.int32(1), 9 - i)
        cnt = jnp.sum((eq & (kiota < t)).astype(jnp.int32),
                      axis=1, keepdims=True)
        return jnp.where(cnt <= r, t, j)
    chosen = jax.lax.fori_loop(0, 10, istep, jnp.zeros((T, 1), jnp.int32))

    code = jnp.where(nb_ref[0] != 0, i2, chosen)
    tbase = pl.program_id(0) * T + K
    tiota = jax.lax.broadcasted_iota(jnp.int32, (T, 1), 0)
    out_ref[0] = jnp.where(m_ref[0] != 0, code, tbase + tiota)


def _select_codes(zf, cb_t, csq_row, zsq3, nb3, m3, T):
    N = zf.shape[0]
    NB = N // T
    return pl.pallas_call(
        functools.partial(_select_body, T=T),
        grid=(NB,),
        in_specs=[pl.BlockSpec((T, D), lambda i: (i, 0)),
                  pl.BlockSpec((D, K), lambda i: (0, 0)),
                  pl.BlockSpec((1, K), lambda i: (0, 0)),
                  pl.BlockSpec((1, T, 1), lambda i: (i, 0, 0)),
                  pl.BlockSpec((1, T, 1), lambda i: (i, 0, 0)),
                  pl.BlockSpec((1, T, 1), lambda i: (i, 0, 0))],
        out_specs=pl.BlockSpec((1, T, 1), lambda i: (i, 0, 0)),
        out_shape=jax.ShapeDtypeStruct((NB, T, 1), jnp.int32),
    )(zf, cb_t, csq_row, zsq3, nb3, m3)


def _make_sc_gather(B, CHUNK):
    mesh = plsc.VectorSubcoreMesh(core_axis_name="c", subcore_axis_name="s")
    b_per_w = B // 32

    @functools.partial(
        pl.kernel, mesh=mesh,
        out_type=jax.ShapeDtypeStruct((B, D), jnp.float32),
        scratch_types=[pltpu.VMEM((CHUNK,), jnp.int32),
                       pltpu.VMEM((CHUNK, D), jnp.float32),
                       pltpu.SemaphoreType.DMA],
    )
    def k(table_hbm, idx_hbm, out_hbm, idx_v, rows_v, sem):
        wid = lax.axis_index("s") * 2 + lax.axis_index("c")
        base = wid * b_per_w
        for j in range(b_per_w // CHUNK):
            off = base + j * CHUNK
            pltpu.sync_copy(idx_hbm.at[pl.ds(off, CHUNK)], idx_v)
            pltpu.async_copy(table_hbm.at[idx_v], rows_v, sem).wait()
            pltpu.sync_copy(rows_v, out_hbm.at[pl.ds(off, CHUNK)])

    return k


def _one_scale(q, cb, mask, rng_key, T):
    B, Dd, H, W = q.shape
    N = B * H * W
    NB = N // T
    ztr = jnp.transpose(q, (0, 2, 3, 1))
    zf = ztr.reshape(-1, Dd)
    ztr_b, zf_b, cb_b = jax.lax.optimization_barrier((ztr, zf, cb))
    if H * W >= 8192:
        zsq = jax.lax.optimization_barrier(
            jnp.sum(zf_b * zf_b, axis=1, keepdims=True))
    else:
        zsq = jax.lax.optimization_barrier(
            jnp.sum(ztr_b * ztr_b, axis=3)).reshape(-1, 1)
    csq = jax.lax.optimization_barrier(jnp.sum(cb_b * cb_b, axis=1))
    use_nb = jax.random.uniform(rng_key, (zf.shape[0],)) < NEIGHBOR_PROB

    zsq3 = zsq.reshape(NB, T, 1)
    nb3 = use_nb.astype(jnp.int32).reshape(NB, T, 1)
    m3 = mask.astype(jnp.int32).reshape(NB, T, 1)
    codes = _select_codes(zf, cb.T, csq.reshape(1, K), zsq3, nb3, m3,
                          T).reshape(N)
    table = jnp.concatenate([cb, zf], axis=0)
    out_tok = _make_sc_gather(N, 128)(table, codes)
    return out_tok.reshape(B, H, W, Dd).transpose(0, 3, 1, 2)


def kernel(q_fine, q_coarse, M, cb_fine, cb_coarse):
    mask_f, mask_c = _project_masks(M)
    key = jax.random.key(42)
    kf, kc = jax.random.split(key)
    q_coarse_a = _one_scale(q_coarse, cb_coarse, mask_c, kc, 512)
    q_fine_a = _one_scale(q_fine, cb_fine, mask_f, kf, 512)
    return (q_fine_a, q_coarse_a)
